# baseline (speedup 1.0000x reference)
import functools

import jax
import jax.numpy as jnp
from jax import lax
from jax.experimental import pallas as pl
from jax.experimental.pallas import tpu as pltpu
from jax.experimental.pallas import tpu_sc as plsc

_NUM_BUCKETS = 32
_MAX_DISTANCE = 128
_WIN = 2304


def _bucket_kernel(bucket_ref, *, length):
    rows, cols = bucket_ref.shape
    j = (
        jax.lax.broadcasted_iota(jnp.int32, (rows, cols), 0) * cols
        + jax.lax.broadcasted_iota(jnp.int32, (rows, cols), 1)
    )
    n = (length - 1) - j
    half = _NUM_BUCKETS // 2
    ret = jnp.where(n < 0, half, 0)
    n_abs = jnp.abs(n)
    max_exact = half // 2
    n_f = n_abs.astype(jnp.float32)
    val_if_large = max_exact + (
        jnp.log(n_f / max_exact + jnp.finfo(jnp.float32).eps)
        / jnp.log(_MAX_DISTANCE / max_exact)
        * (half - max_exact)
    ).astype(jnp.int32)
    val_if_large = jnp.minimum(val_if_large, half - 1)
    bucket_ref[...] = ret + jnp.where(n_abs < max_exact, n_abs, val_if_large)


def _make_sc_gather(num_j, diag_w):
    info = plsc.get_sparse_core_info()
    num_workers = info.num_cores * info.num_subcores
    j_per_w = num_j // num_workers
    groups = j_per_w // 16
    mesh = plsc.VectorSubcoreMesh(core_axis_name="c", subcore_axis_name="s")

    @functools.partial(
        pl.kernel,
        mesh=mesh,
        out_type=jax.ShapeDtypeStruct((16, diag_w), jnp.float32),
        scratch_types=[
            pltpu.VMEM((j_per_w,), jnp.int32),
            pltpu.VMEM((512,), jnp.float32),
            pltpu.VMEM((16, j_per_w), jnp.float32),
        ],
        compiler_params=pltpu.CompilerParams(needs_layout_passes=False),
    )
    def sc_gather(table_hbm, idx_hbm, out_hbm, idx_v, table_v, cols_v):
        wid = lax.axis_index("s") * info.num_cores + lax.axis_index("c")
        pltpu.sync_copy(idx_hbm.at[pl.ds(wid * j_per_w, j_per_w)], idx_v)
        pltpu.sync_copy(table_hbm, table_v)
        lane = lax.iota(jnp.int32, 16)
        for g in range(groups):
            bvec = idx_v[pl.ds(g * 16, 16)]
            jvec = lane + (g * 16)
            for h in range(16):
                vals = plsc.load_gather(table_v, [bvec * 16 + h])
                plsc.store_scatter(
                    cols_v, [jnp.full((16,), h, jnp.int32), jvec], vals
                )
        pltpu.sync_copy(
            cols_v, out_hbm.at[:, pl.ds(wid * j_per_w, j_per_w)]
        )

    return sc_gather


def _expand_kernel(diag_ref, out_ref, *, tq):
    length = out_ref.shape[3]
    base = pl.multiple_of(length - tq * (pl.program_id(1) + 1), 128)
    w = diag_ref[0, :, pl.ds(base, _WIN)]
    wb = jnp.broadcast_to(w, (8, _WIN))
    for g in range(tq // 8):
        rolled = pltpu.roll(
            wb, _WIN - (tq - 1) + 8 * g, axis=1, stride=1, stride_axis=0
        )
        out_ref[0, 0, pl.ds(8 * g, 8), :] = rolled[:, :length]


def kernel(inputs_q, embed_table):
    batch, length, _ = inputs_q.shape
    heads = embed_table.shape[1]

    diag_w = ((length - 1) // 128 * 128) + _WIN

    num_j = 4096
    bucket_idx = pl.pallas_call(
        functools.partial(_bucket_kernel, length=length),
        out_shape=jax.ShapeDtypeStruct((num_j // 128, 128), jnp.int32),
    )()

    table_pad = jnp.zeros((_NUM_BUCKETS, 16), jnp.float32)
    table_pad = table_pad.at[:, :heads].set(embed_table)

    diag = _make_sc_gather(num_j, diag_w)(
        table_pad.reshape(-1), bucket_idx.reshape(-1)
    )

    tq = 256
    out = pl.pallas_call(
        functools.partial(_expand_kernel, tq=tq),
        grid=(heads, length // tq),
        in_specs=[
            pl.BlockSpec((1, 1, diag_w), lambda h, i: (h, 0, 0)),
        ],
        out_specs=pl.BlockSpec(
            (1, 1, tq, length), lambda h, i: (0, h, i, 0)
        ),
        out_shape=jax.ShapeDtypeStruct((batch, heads, length, length), jnp.float32),
        compiler_params=pltpu.CompilerParams(
            dimension_semantics=("arbitrary", "arbitrary")
        ),
    )(diag.reshape(16, 1, diag_w))
    return out


# speedup vs baseline: 1.2215x; 1.2215x over previous
import functools

import numpy as np

import jax
import jax.numpy as jnp
from jax import lax
from jax.experimental import pallas as pl
from jax.experimental.pallas import tpu as pltpu
from jax.experimental.pallas import tpu_sc as plsc

_NUM_BUCKETS = 32
_MAX_DISTANCE = 128
_WIN = 2304


def _bucket_thresholds():
    n = np.arange(8, 8192)
    v = (
        np.log(n.astype(np.float32) / np.float32(8.0) + np.float32(np.finfo(np.float32).eps))
        / np.float32(np.log(np.float32(16.0)))
        * np.float32(8.0)
    ).astype(np.int32)
    v = np.minimum(8 + v, 15)
    return [int(n[v >= 8 + m][0]) for m in range(1, 8)]


_THRESHOLDS = _bucket_thresholds()


def _make_sc_gather(num_j, diag_w, length):
    info = plsc.get_sparse_core_info()
    num_workers = info.num_cores * info.num_subcores
    j_per_w = num_j // num_workers
    groups = j_per_w // 16
    mesh = plsc.VectorSubcoreMesh(core_axis_name="c", subcore_axis_name="s")

    @functools.partial(
        pl.kernel,
        mesh=mesh,
        out_type=jax.ShapeDtypeStruct((16, diag_w), jnp.float32),
        scratch_types=[
            pltpu.VMEM((512,), jnp.float32),
            pltpu.VMEM((16, j_per_w), jnp.float32),
        ],
        compiler_params=pltpu.CompilerParams(needs_layout_passes=False),
    )
    def sc_gather(table_hbm, out_hbm, table_v, cols_v):
        wid = lax.axis_index("s") * info.num_cores + lax.axis_index("c")
        pltpu.sync_copy(table_hbm, table_v)
        lane = lax.iota(jnp.int32, 16)
        for g in range(groups):
            jvec = lane + (g * 16)
            j = jvec + wid * j_per_w
            n = (length - 1) - j
            na = jnp.abs(n)
            large = jnp.full((16,), 8, jnp.int32)
            for t in _THRESHOLDS:
                large = large + jnp.where(na >= t, 1, 0).astype(jnp.int32)
            bvec = jnp.where(na < 8, na, large) + jnp.where(n < 0, 16, 0)
            for h in range(16):
                vals = plsc.load_gather(table_v, [bvec * 16 + h])
                plsc.store_scatter(
                    cols_v, [jnp.full((16,), h, jnp.int32), jvec], vals
                )
        pltpu.sync_copy(
            cols_v, out_hbm.at[:, pl.ds(wid * j_per_w, j_per_w)]
        )

    return sc_gather


def _expand_kernel(diag_ref, out_ref, *, tq):
    heads = out_ref.shape[1]
    length = out_ref.shape[3]
    base = pl.multiple_of(length - tq * (pl.program_id(0) + 1), 128)
    for h in range(heads):
        w = diag_ref[pl.ds(h, 1), pl.ds(base, _WIN)]
        wb = jnp.broadcast_to(w, (8, _WIN))
        for g in range(tq // 8):
            rolled = pltpu.roll(
                wb, _WIN - (tq - 1) + 8 * g, axis=1, stride=1, stride_axis=0
            )
            out_ref[0, h, pl.ds(8 * g, 8), :] = rolled[:, :length]


def kernel(inputs_q, embed_table):
    batch, length, _ = inputs_q.shape
    heads = embed_table.shape[1]

    diag_w = ((length - 1) // 128 * 128) + _WIN

    num_j = 4096

    table_pad = jnp.zeros((_NUM_BUCKETS, 16), jnp.float32)
    table_pad = table_pad.at[:, :heads].set(embed_table)

    diag = _make_sc_gather(num_j, diag_w, length)(table_pad.reshape(-1))

    tq = 256
    out = pl.pallas_call(
        functools.partial(_expand_kernel, tq=tq),
        grid=(length // tq,),
        in_specs=[pl.BlockSpec((16, diag_w), lambda i: (0, 0))],
        out_specs=pl.BlockSpec((1, heads, tq, length), lambda i: (0, 0, i, 0)),
        out_shape=jax.ShapeDtypeStruct((batch, heads, length, length), jnp.float32),
        compiler_params=pltpu.CompilerParams(
            dimension_semantics=("arbitrary",)
        ),
    )(diag)
    return out
